# Initial kernel scaffold; baseline (speedup 1.0000x reference)
import jax
import jax.numpy as jnp
from jax.experimental import pallas as pl


def kernel(xyz, f1_W1a, f1_b1a, f1_W1b, f1_b1b, f1_W2a, f1_b2a, f1_W2b, f1_b2b, f1_Wf, f2_W1a, f2_b1a, f2_W1b, f2_b1b, f2_W2a, f2_b2a, f2_W2b, f2_b2b, f2_Wf, f3_W1a, f3_b1a, f3_W1b, f3_b1b, f3_W2a, f3_b2a, f3_W2b, f3_b2b, f3_Wf, fc1_W, fc1_b, bn1_g, bn1_b, fc2_W, fc2_b, bn2_g, bn2_b, fc3_W, fc3_b):
    raise NotImplementedError("write your pallas kernel here")


# speedup vs baseline: 3.2286x; 3.2286x over previous
import jax
import jax.numpy as jnp
import numpy as np
from jax import lax
from jax.experimental import pallas as pl

_FP = np.array([[1.0, 1.0, 1.0], [1.0, 1.0, -1.0], [1.0, -1.0, 1.0], [1.0, -1.0, -1.0],
                [-1.0, 1.0, 1.0], [-1.0, 1.0, -1.0], [-1.0, -1.0, 1.0], [-1.0, -1.0, -1.0],
                [0.0, 0.0, 0.0]], dtype=np.float32)

_HI = jax.lax.Precision.HIGHEST


def _tdot(a, b):
    return lax.dot_general(a, b, (((0,), (0,)), ((), ())),
                           precision=_HI, preferred_element_type=jnp.float32)


def _mdot(a, b):
    return lax.dot_general(a, b, (((1,), (0,)), ((), ())),
                           precision=_HI, preferred_element_type=jnp.float32)


def _fpac_loop(ptsT, featT, centersT, fps, nsample,
               W1a, b1a, W1b, b1b, W2a, b2a, W2b, b2b, Wf):
    P = ptsT.shape[1]
    C = centersT.shape[1]
    F2 = W2b.shape[1]

    ones3 = jnp.ones((3, 1), jnp.float32)
    pn2_col = _tdot(ptsT * ptsT, ones3)
    cn2_row = jnp.sum(centersT * centersT, axis=0, keepdims=True)
    cross = _tdot(ptsT, centersT)
    d2 = pn2_col + cn2_row - 2.0 * cross

    fp2_col = jnp.sum(fps * fps, axis=1, keepdims=True)
    iota = lax.broadcasted_iota(jnp.int32, (P, C), 0)

    def step(_, carry):
        d2, acc = carry
        m = jnp.min(d2, axis=0, keepdims=True)
        elig = d2 <= m
        first = jnp.min(jnp.where(elig, iota, jnp.int32(P)),
                        axis=0, keepdims=True)
        oh = jnp.where(iota == first, 1.0, 0.0)
        d2 = d2 + oh * 1e30

        gx = _mdot(ptsT, oh)
        gf = _mdot(featT, oh)
        relT = gx - centersT

        dd = fp2_col + jnp.sum(relT * relT, axis=0, keepdims=True) \
            - 2.0 * _mdot(fps, relT)
        dd = -dd
        mm = jnp.max(dd, axis=0, keepdims=True)
        e = jnp.exp(dd - mm)
        w = e / jnp.sum(e, axis=0, keepdims=True)
        renc = _tdot(fps, w)

        a1 = jnp.maximum(_tdot(W1a, renc) + b1a, 0.0)
        h = _tdot(W1b, a1) + b1b
        a2 = jnp.maximum(_tdot(W2a, h) + b2a, 0.0)
        kern = _tdot(W2b, a2) + b2b
        fproj = _tdot(Wf, gf)
        acc = jnp.maximum(acc, kern * fproj)
        return d2, acc

    acc0 = jnp.full((F2, C), -jnp.inf, jnp.float32)
    _, out = lax.fori_loop(0, nsample, step, (d2, acc0))
    return out


def _select_centers(ptsT, stride, C):
    P = ptsT.shape[1]
    rows = lax.broadcasted_iota(jnp.int32, (P, C), 0)
    cols = lax.broadcasted_iota(jnp.int32, (P, C), 1)
    E = jnp.where(rows == stride * cols, 1.0, 0.0)
    return _mdot(ptsT, E)


def _body(*refs):
    (xyz_ref,
     f1_W1a, f1_b1a, f1_W1b, f1_b1b, f1_W2a, f1_b2a, f1_W2b, f1_b2b, f1_Wf,
     f2_W1a, f2_b1a, f2_W1b, f2_b1b, f2_W2a, f2_b2a, f2_W2b, f2_b2b, f2_Wf,
     f3_W1a, f3_b1a, f3_W1b, f3_b1b, f3_W2a, f3_b2a, f3_W2b, f3_b2b, f3_Wf,
     fc1_W, fc1_b, bn1_g, bn1_b, fc2_W, fc2_b, bn2_g, bn2_b, fc3_W, fc3_b,
     fps1, fps2, fps3, out_ref) = refs

    coordsT = xyz_ref[0, 0:3, :]
    normT = xyz_ref[0, 3:6, :]

    c1 = _select_centers(coordsT, 2, 512)
    o1 = _fpac_loop(coordsT, normT, c1, fps1[...], 32,
                    f1_W1a[...], f1_b1a[...], f1_W1b[...], f1_b1b[...],
                    f1_W2a[...], f1_b2a[...], f1_W2b[...], f1_b2b[...],
                    f1_Wf[...])

    c2 = _select_centers(c1, 4, 128)
    o2 = _fpac_loop(c1, o1, c2, fps2[...], 64,
                    f2_W1a[...], f2_b1a[...], f2_W1b[...], f2_b1b[...],
                    f2_W2a[...], f2_b2a[...], f2_W2b[...], f2_b2b[...],
                    f2_Wf[...])

    fps3v = fps3[...]
    fp2_col = jnp.sum(fps3v * fps3v, axis=1, keepdims=True)
    relT = c2
    dd = fp2_col + jnp.sum(relT * relT, axis=0, keepdims=True) \
        - 2.0 * _mdot(fps3v, relT)
    dd = -dd
    mm = jnp.max(dd, axis=0, keepdims=True)
    e = jnp.exp(dd - mm)
    w = e / jnp.sum(e, axis=0, keepdims=True)
    renc = _tdot(fps3v, w)
    a1 = jnp.maximum(_tdot(f3_W1a[...], renc) + f3_b1a[...], 0.0)
    h = _tdot(f3_W1b[...], a1) + f3_b1b[...]
    a2 = jnp.maximum(_tdot(f3_W2a[...], h) + f3_b2a[...], 0.0)
    kern = _tdot(f3_W2b[...], a2) + f3_b2b[...]
    fproj = _tdot(f3_Wf[...], o2)
    g = jnp.max(kern * fproj, axis=1, keepdims=True)

    inv = jnp.float32(1.0 / np.sqrt(1.0 + 1e-5))
    x = _tdot(g, fc1_W[...]) + fc1_b[...]
    x = jnp.maximum(x * inv * bn1_g[...] + bn1_b[...], 0.0)
    x = _mdot(x, fc2_W[...]) + fc2_b[...]
    x = jnp.maximum(x * inv * bn2_g[...] + bn2_b[...], 0.0)
    x = _mdot(x, fc3_W[...]) + fc3_b[...]
    xm = jnp.max(x, axis=1, keepdims=True)
    lse = jnp.log(jnp.sum(jnp.exp(x - xm), axis=1, keepdims=True)) + xm
    out_ref[0] = x - lse


def kernel(xyz, f1_W1a, f1_b1a, f1_W1b, f1_b1b, f1_W2a, f1_b2a, f1_W2b, f1_b2b, f1_Wf,
           f2_W1a, f2_b1a, f2_W1b, f2_b1b, f2_W2a, f2_b2a, f2_W2b, f2_b2b, f2_Wf,
           f3_W1a, f3_b1a, f3_W1b, f3_b1b, f3_W2a, f3_b2a, f3_W2b, f3_b2b, f3_Wf,
           fc1_W, fc1_b, bn1_g, bn1_b, fc2_W, fc2_b, bn2_g, bn2_b, fc3_W, fc3_b):
    B = xyz.shape[0]
    col = lambda v: v.reshape(-1, 1)
    row = lambda v: v.reshape(1, -1)
    operands = [
        xyz,
        f1_W1a, col(f1_b1a), f1_W1b, col(f1_b1b),
        f1_W2a, col(f1_b2a), f1_W2b, col(f1_b2b), f1_Wf,
        f2_W1a, col(f2_b1a), f2_W1b, col(f2_b1b),
        f2_W2a, col(f2_b2a), f2_W2b, col(f2_b2b), f2_Wf,
        f3_W1a, col(f3_b1a), f3_W1b, col(f3_b1b),
        f3_W2a, col(f3_b2a), f3_W2b, col(f3_b2b), f3_Wf,
        fc1_W, row(fc1_b), row(bn1_g), row(bn1_b),
        fc2_W, row(fc2_b), row(bn2_g), row(bn2_b),
        fc3_W, row(fc3_b),
        jnp.asarray(_FP * 0.2), jnp.asarray(_FP * 0.4), jnp.asarray(_FP * 0.8),
    ]

    def const_spec(op):
        nd = op.ndim
        return pl.BlockSpec(op.shape, lambda b, _nd=nd: (0,) * _nd)

    in_specs = [pl.BlockSpec((1, 6, 1024), lambda b: (b, 0, 0))]
    in_specs += [const_spec(op) for op in operands[1:]]

    out = pl.pallas_call(
        _body,
        grid=(B,),
        in_specs=in_specs,
        out_specs=pl.BlockSpec((1, 1, 40), lambda b: (b, 0, 0)),
        out_shape=jax.ShapeDtypeStruct((B, 1, 40), jnp.float32),
    )(*operands)
    s3 = jnp.zeros((B, 1, 3), jnp.float32)
    return out.reshape(B, 40), s3
